# Initial kernel scaffold; baseline (speedup 1.0000x reference)
import jax
import jax.numpy as jnp
from jax.experimental import pallas as pl


def kernel(input_bags, emb_table, W, b):
    raise NotImplementedError("write your pallas kernel here")


# speedup vs baseline: 11.9577x; 11.9577x over previous
import functools

import jax
import jax.numpy as jnp
from jax import lax
from jax.experimental import pallas as pl
from jax.experimental.pallas import tpu as pltpu
from jax.experimental.pallas import tpu_sc as plsc

B = 4096
L = 200
D = 128
C = 1000

NC, NS = 2, 16
NW = NC * NS
BPW = B // NW
C0, C1 = 104, 96

_LANES = 16
_G = D // _LANES


def _accum_rows(buf, n, acc):
    def body(r, acc):
        return tuple(acc[g] + buf[r, pl.ds(g * _LANES, _LANES)]
                     for g in range(_G))
    return lax.fori_loop(0, n, body, acc)


def _bag_sum_body(bags_hbm, table_hbm, out_hbm,
                  idx_v, bufA0, bufA1, bufB0, bufB1, outs_v,
                  semA0, semA1, semB0, semB1):
    wid = lax.axis_index("s") * NC + lax.axis_index("c")
    base = wid * BPW

    pltpu.sync_copy(bags_hbm.at[pl.ds(base * L, BPW * L)], idx_v)

    def issue(b, buf0, buf1, sem0, sem1):
        pltpu.async_copy(table_hbm.at[idx_v.at[pl.ds(b * L, C0)]], buf0, sem0)
        pltpu.async_copy(table_hbm.at[idx_v.at[pl.ds(b * L + C0, C1)]],
                         buf1, sem1)

    def drain_reduce(b, buf0, buf1, sem0, sem1):
        acc = tuple(jnp.zeros((_LANES,), jnp.float32) for _ in range(_G))
        pltpu.make_async_copy(
            table_hbm.at[idx_v.at[pl.ds(b * L, C0)]], buf0, sem0).wait()
        acc = _accum_rows(buf0, C0, acc)
        pltpu.make_async_copy(
            table_hbm.at[idx_v.at[pl.ds(b * L + C0, C1)]], buf1, sem1).wait()
        acc = _accum_rows(buf1, C1, acc)
        for g in range(_G):
            outs_v[b, pl.ds(g * _LANES, _LANES)] = acc[g]

    issue(0, bufA0, bufA1, semA0, semA1)

    def pair_body(i, _):
        be = 2 * i
        bo = 2 * i + 1
        issue(bo, bufB0, bufB1, semB0, semB1)
        drain_reduce(be, bufA0, bufA1, semA0, semA1)

        @pl.when(be + 2 < BPW)
        def _():
            issue(be + 2, bufA0, bufA1, semA0, semA1)

        drain_reduce(bo, bufB0, bufB1, semB0, semB1)
        return 0

    lax.fori_loop(0, BPW // 2, pair_body, 0)

    pltpu.sync_copy(outs_v, out_hbm.at[pl.ds(base, BPW)])


@jax.jit
def _sc_bag_sum(input_bags, emb_table):
    mesh = plsc.VectorSubcoreMesh(core_axis_name="c", subcore_axis_name="s")
    return pl.kernel(
        _bag_sum_body,
        out_type=jax.ShapeDtypeStruct((B, D), jnp.float32),
        mesh=mesh,
        scratch_types=[
            pltpu.VMEM((BPW * L,), jnp.int32),
            pltpu.VMEM((C0, D), jnp.float32),
            pltpu.VMEM((C1, D), jnp.float32),
            pltpu.VMEM((C0, D), jnp.float32),
            pltpu.VMEM((C1, D), jnp.float32),
            pltpu.VMEM((BPW, D), jnp.float32),
            pltpu.SemaphoreType.DMA,
            pltpu.SemaphoreType.DMA,
            pltpu.SemaphoreType.DMA,
            pltpu.SemaphoreType.DMA,
        ],
    )(input_bags, emb_table)


def _head_body(h_ref, wt_ref, b_ref, o_ref):
    h = h_ref[...] * (1.0 / L)
    logits = jnp.dot(h, wt_ref[...],
                     preferred_element_type=jnp.float32) + b_ref[...]
    m = jnp.max(logits, axis=1, keepdims=True)
    s = logits - m
    lse = jnp.log(jnp.sum(jnp.exp(s), axis=1, keepdims=True))
    o_ref[...] = s - lse


@jax.jit
def _tc_head(hidden_sums, Wt, b2):
    blk = 256
    return pl.pallas_call(
        _head_body,
        grid=(B // blk,),
        in_specs=[
            pl.BlockSpec((blk, D), lambda i: (i, 0)),
            pl.BlockSpec((D, C), lambda i: (0, 0)),
            pl.BlockSpec((1, C), lambda i: (0, 0)),
        ],
        out_specs=pl.BlockSpec((blk, C), lambda i: (i, 0)),
        out_shape=jax.ShapeDtypeStruct((B, C), jnp.float32),
    )(hidden_sums, Wt, b2)


def kernel(input_bags, emb_table, W, b):
    sums = _sc_bag_sum(input_bags.astype(jnp.int32).reshape(-1), emb_table)
    return _tc_head(sums, W.T, b.reshape(1, C))


# speedup vs baseline: 11.9835x; 1.0022x over previous
import functools

import jax
import jax.numpy as jnp
from jax import lax
from jax.experimental import pallas as pl
from jax.experimental.pallas import tpu as pltpu
from jax.experimental.pallas import tpu_sc as plsc

B = 4096
L = 200
D = 128
C = 1000

NC, NS = 2, 16
NW = NC * NS
BPW = B // NW
C0, C1 = 104, 96

_LANES = 16
_G = D // _LANES


_UNROLL = 4


def _accum_rows(buf, n, acc):
    def body(i, acc):
        r0 = i * _UNROLL
        for u in range(_UNROLL):
            acc = tuple(acc[g] + buf[r0 + u, pl.ds(g * _LANES, _LANES)]
                        for g in range(_G))
        return acc
    return lax.fori_loop(0, n // _UNROLL, body, acc)


def _bag_sum_body(bags_hbm, table_hbm, out_hbm,
                  idx_v, bufA0, bufA1, bufB0, bufB1, outs_v,
                  semA0, semA1, semB0, semB1):
    wid = lax.axis_index("s") * NC + lax.axis_index("c")
    base = wid * BPW

    pltpu.sync_copy(bags_hbm.at[pl.ds(base * L, BPW * L)], idx_v)

    def issue(b, buf0, buf1, sem0, sem1):
        pltpu.async_copy(table_hbm.at[idx_v.at[pl.ds(b * L, C0)]], buf0, sem0)
        pltpu.async_copy(table_hbm.at[idx_v.at[pl.ds(b * L + C0, C1)]],
                         buf1, sem1)

    def drain_reduce(b, buf0, buf1, sem0, sem1):
        acc = tuple(jnp.zeros((_LANES,), jnp.float32) for _ in range(_G))
        pltpu.make_async_copy(
            table_hbm.at[idx_v.at[pl.ds(b * L, C0)]], buf0, sem0).wait()
        acc = _accum_rows(buf0, C0, acc)
        pltpu.make_async_copy(
            table_hbm.at[idx_v.at[pl.ds(b * L + C0, C1)]], buf1, sem1).wait()
        acc = _accum_rows(buf1, C1, acc)
        for g in range(_G):
            outs_v[b, pl.ds(g * _LANES, _LANES)] = acc[g]

    issue(0, bufA0, bufA1, semA0, semA1)

    def pair_body(i, _):
        be = 2 * i
        bo = 2 * i + 1
        issue(bo, bufB0, bufB1, semB0, semB1)
        drain_reduce(be, bufA0, bufA1, semA0, semA1)

        @pl.when(be + 2 < BPW)
        def _():
            issue(be + 2, bufA0, bufA1, semA0, semA1)

        drain_reduce(bo, bufB0, bufB1, semB0, semB1)
        return 0

    lax.fori_loop(0, BPW // 2, pair_body, 0)

    pltpu.sync_copy(outs_v, out_hbm.at[pl.ds(base, BPW)])


@jax.jit
def _sc_bag_sum(input_bags, emb_table):
    mesh = plsc.VectorSubcoreMesh(core_axis_name="c", subcore_axis_name="s")
    return pl.kernel(
        _bag_sum_body,
        out_type=jax.ShapeDtypeStruct((B, D), jnp.float32),
        mesh=mesh,
        scratch_types=[
            pltpu.VMEM((BPW * L,), jnp.int32),
            pltpu.VMEM((C0, D), jnp.float32),
            pltpu.VMEM((C1, D), jnp.float32),
            pltpu.VMEM((C0, D), jnp.float32),
            pltpu.VMEM((C1, D), jnp.float32),
            pltpu.VMEM((BPW, D), jnp.float32),
            pltpu.SemaphoreType.DMA,
            pltpu.SemaphoreType.DMA,
            pltpu.SemaphoreType.DMA,
            pltpu.SemaphoreType.DMA,
        ],
    )(input_bags, emb_table)


def _head_body(h_ref, wt_ref, b_ref, o_ref):
    h = h_ref[...] * (1.0 / L)
    logits = jnp.dot(h, wt_ref[...],
                     preferred_element_type=jnp.float32) + b_ref[...]
    m = jnp.max(logits, axis=1, keepdims=True)
    s = logits - m
    lse = jnp.log(jnp.sum(jnp.exp(s), axis=1, keepdims=True))
    o_ref[...] = s - lse


@jax.jit
def _tc_head(hidden_sums, Wt, b2):
    blk = 256
    return pl.pallas_call(
        _head_body,
        grid=(B // blk,),
        in_specs=[
            pl.BlockSpec((blk, D), lambda i: (i, 0)),
            pl.BlockSpec((D, C), lambda i: (0, 0)),
            pl.BlockSpec((1, C), lambda i: (0, 0)),
        ],
        out_specs=pl.BlockSpec((blk, C), lambda i: (i, 0)),
        out_shape=jax.ShapeDtypeStruct((B, C), jnp.float32),
    )(hidden_sums, Wt, b2)


def kernel(input_bags, emb_table, W, b):
    sums = _sc_bag_sum(input_bags.astype(jnp.int32).reshape(-1), emb_table)
    return _tc_head(sums, W.T, b.reshape(1, C))


# speedup vs baseline: 14.0374x; 1.1714x over previous
import functools

import jax
import jax.numpy as jnp
from jax import lax
from jax.experimental import pallas as pl
from jax.experimental.pallas import tpu as pltpu
from jax.experimental.pallas import tpu_sc as plsc

B = 4096
L = 200
D = 128
C = 1000

NC, NS = 2, 16
NW = NC * NS
BPW = B // NW
C0, C1 = 104, 96

_LANES = 16
_G = D // _LANES


_UNROLL = 4


def _accum_rows(buf, n, acc):
    def body(i, acc):
        r0 = i * _UNROLL
        for u in range(_UNROLL):
            acc = tuple(acc[g] + buf[r0 + u, pl.ds(g * _LANES, _LANES)]
                        for g in range(_G))
        return acc
    return lax.fori_loop(0, n // _UNROLL, body, acc)


_NSETS = 3


def _bag_sum_body(bags_hbm, table_hbm, out_hbm,
                  idx_v, buf00, buf01, buf10, buf11, buf20, buf21, outs_v,
                  sem00, sem01, sem10, sem11, sem20, sem21):
    wid = lax.axis_index("s") * NC + lax.axis_index("c")
    base = wid * BPW
    bufs = ((buf00, buf01), (buf10, buf11), (buf20, buf21))
    sems = ((sem00, sem01), (sem10, sem11), (sem20, sem21))

    pltpu.sync_copy(bags_hbm.at[pl.ds(base * L, BPW * L)], idx_v)

    def issue(b, k):
        pltpu.async_copy(table_hbm.at[idx_v.at[pl.ds(b * L, C0)]],
                         bufs[k][0], sems[k][0])
        pltpu.async_copy(table_hbm.at[idx_v.at[pl.ds(b * L + C0, C1)]],
                         bufs[k][1], sems[k][1])

    def drain_reduce(b, k):
        acc = tuple(jnp.zeros((_LANES,), jnp.float32) for _ in range(_G))
        pltpu.make_async_copy(
            table_hbm.at[idx_v.at[pl.ds(b * L, C0)]],
            bufs[k][0], sems[k][0]).wait()
        acc = _accum_rows(bufs[k][0], C0, acc)
        pltpu.make_async_copy(
            table_hbm.at[idx_v.at[pl.ds(b * L + C0, C1)]],
            bufs[k][1], sems[k][1]).wait()
        acc = _accum_rows(bufs[k][1], C1, acc)
        for g in range(_G):
            outs_v[b, pl.ds(g * _LANES, _LANES)] = acc[g]

    for k in range(_NSETS):
        issue(k, k)

    def group_body(i, _):
        for k in range(_NSETS):
            b = _NSETS * i + k
            drain_reduce(b, k)

            @pl.when(b + _NSETS < BPW)
            def _():
                issue(b + _NSETS, k)
        return 0

    ngroups = BPW // _NSETS
    lax.fori_loop(0, ngroups, group_body, 0)
    for k in range(BPW - _NSETS * ngroups):
        drain_reduce(_NSETS * ngroups + k, k)

    pltpu.sync_copy(outs_v, out_hbm.at[pl.ds(base, BPW)])


@jax.jit
def _sc_bag_sum(input_bags, emb_table):
    mesh = plsc.VectorSubcoreMesh(core_axis_name="c", subcore_axis_name="s")
    return pl.kernel(
        _bag_sum_body,
        out_type=jax.ShapeDtypeStruct((B, D), jnp.float32),
        mesh=mesh,
        scratch_types=(
            [pltpu.VMEM((BPW * L,), jnp.int32)]
            + [pltpu.VMEM((n, D), jnp.float32)
               for _ in range(_NSETS) for n in (C0, C1)]
            + [pltpu.VMEM((BPW, D), jnp.float32)]
            + [pltpu.SemaphoreType.DMA] * (2 * _NSETS)
        ),
    )(input_bags, emb_table)


def _head_body(h_ref, wt_ref, b_ref, o_ref):
    h = h_ref[...] * (1.0 / L)
    logits = jnp.dot(h, wt_ref[...],
                     preferred_element_type=jnp.float32) + b_ref[...]
    m = jnp.max(logits, axis=1, keepdims=True)
    s = logits - m
    lse = jnp.log(jnp.sum(jnp.exp(s), axis=1, keepdims=True))
    o_ref[...] = s - lse


@jax.jit
def _tc_head(hidden_sums, Wt, b2):
    blk = 256
    return pl.pallas_call(
        _head_body,
        grid=(B // blk,),
        in_specs=[
            pl.BlockSpec((blk, D), lambda i: (i, 0)),
            pl.BlockSpec((D, C), lambda i: (0, 0)),
            pl.BlockSpec((1, C), lambda i: (0, 0)),
        ],
        out_specs=pl.BlockSpec((blk, C), lambda i: (i, 0)),
        out_shape=jax.ShapeDtypeStruct((B, C), jnp.float32),
    )(hidden_sums, Wt, b2)


def kernel(input_bags, emb_table, W, b):
    sums = _sc_bag_sum(input_bags.astype(jnp.int32).reshape(-1), emb_table)
    return _tc_head(sums, W.T, b.reshape(1, C))


# speedup vs baseline: 15.2044x; 1.0831x over previous
import functools

import jax
import jax.numpy as jnp
from jax import lax
from jax.experimental import pallas as pl
from jax.experimental.pallas import tpu as pltpu
from jax.experimental.pallas import tpu_sc as plsc

B = 4096
L = 200
D = 128
C = 1000

NC, NS = 2, 16
NW = NC * NS
BPW = B // NW
C0, C1 = 104, 96

_LANES = 16
_G = D // _LANES


_UNROLL = 4


def _accum_rows(buf, n, acc):
    def body(i, acc):
        r0 = i * _UNROLL
        for u in range(_UNROLL):
            acc = tuple(acc[g] + buf[r0 + u, pl.ds(g * _LANES, _LANES)]
                        for g in range(_G))
        return acc
    return lax.fori_loop(0, n // _UNROLL, body, acc)


_NSETS = 3


def _bag_sum_body(bags_hbm, table_hbm, out_hbm,
                  idx_v, buf00, buf01, buf10, buf11, buf20, buf21, outs_v,
                  sem00, sem01, sem10, sem11, sem20, sem21):
    wid = lax.axis_index("s") * NC + lax.axis_index("c")
    base = wid * BPW
    bufs = ((buf00, buf01), (buf10, buf11), (buf20, buf21))
    sems = ((sem00, sem01), (sem10, sem11), (sem20, sem21))

    pltpu.sync_copy(bags_hbm.at[pl.ds(base * L, BPW * L)], idx_v)

    def issue(b, k):
        pltpu.async_copy(table_hbm.at[idx_v.at[pl.ds(b * L, C0)]],
                         bufs[k][0], sems[k][0])
        pltpu.async_copy(table_hbm.at[idx_v.at[pl.ds(b * L + C0, C1)]],
                         bufs[k][1], sems[k][1])

    def drain_reduce(b, k):
        acc = tuple(jnp.zeros((_LANES,), jnp.float32) for _ in range(_G))
        pltpu.make_async_copy(
            table_hbm.at[idx_v.at[pl.ds(b * L, C0)]],
            bufs[k][0], sems[k][0]).wait()
        acc = _accum_rows(bufs[k][0], C0, acc)
        pltpu.make_async_copy(
            table_hbm.at[idx_v.at[pl.ds(b * L + C0, C1)]],
            bufs[k][1], sems[k][1]).wait()
        acc = _accum_rows(bufs[k][1], C1, acc)
        for g in range(_G):
            outs_v[b, pl.ds(g * _LANES, _LANES)] = acc[g]

    for k in range(_NSETS):
        issue(k, k)

    def group_body(i, _):
        for k in range(_NSETS):
            b = _NSETS * i + k
            drain_reduce(b, k)

            @pl.when(b + _NSETS < BPW)
            def _():
                issue(b + _NSETS, k)
        return 0

    ngroups = BPW // _NSETS
    lax.fori_loop(0, ngroups, group_body, 0)
    for k in range(BPW - _NSETS * ngroups):
        drain_reduce(_NSETS * ngroups + k, k)

    pltpu.sync_copy(outs_v, out_hbm.at[pl.ds(base, BPW)])


@jax.jit
def _sc_bag_sum(input_bags, emb_table):
    mesh = plsc.VectorSubcoreMesh(core_axis_name="c", subcore_axis_name="s")
    return pl.kernel(
        _bag_sum_body,
        out_type=jax.ShapeDtypeStruct((B, D), jnp.float32),
        mesh=mesh,
        scratch_types=(
            [pltpu.VMEM((BPW * L,), jnp.int32)]
            + [pltpu.VMEM((n, D), jnp.float32)
               for _ in range(_NSETS) for n in (C0, C1)]
            + [pltpu.VMEM((BPW, D), jnp.float32)]
            + [pltpu.SemaphoreType.DMA] * (2 * _NSETS)
        ),
    )(input_bags, emb_table)


def _head_body(h_ref, w_ref, bt_ref, o_ref):
    h = h_ref[...] * (1.0 / L)
    logits = jax.lax.dot_general(
        w_ref[...], h, (((1,), (1,)), ((), ())),
        preferred_element_type=jnp.float32) + bt_ref[...]
    m = jnp.max(logits, axis=0, keepdims=True)
    s = logits - m
    lse = jnp.log(jnp.sum(jnp.exp(s), axis=0, keepdims=True))
    o_ref[...] = s - lse


@jax.jit
def _tc_head(hidden_sums, W, bt):
    blk = 256
    return pl.pallas_call(
        _head_body,
        grid=(B // blk,),
        in_specs=[
            pl.BlockSpec((blk, D), lambda i: (i, 0)),
            pl.BlockSpec((C, D), lambda i: (0, 0)),
            pl.BlockSpec((C, 1), lambda i: (0, 0)),
        ],
        out_specs=pl.BlockSpec((C, blk), lambda i: (0, i)),
        out_shape=jax.ShapeDtypeStruct((C, B), jnp.float32),
    )(hidden_sums, W, bt)


def kernel(input_bags, emb_table, W, b):
    sums = _sc_bag_sum(input_bags.astype(jnp.int32).reshape(-1), emb_table)
    return _tc_head(sums, W, b.reshape(C, 1)).T
